# baseline (speedup 1.0000x reference)
import functools

import jax
import jax.numpy as jnp
from jax import lax
from jax.experimental import pallas as pl
from jax.experimental.pallas import tpu as pltpu
from jax.experimental.pallas import tpu_sc as plsc

R = 4
D = 256
N = 10000
E = 160000
NCORES = 2
NSUB = 16
NW = NCORES * NSUB
OWN = 320
LASTSTART = N - OWN
CH = 1280
GRP2 = CH // 32
NCHUNK = E // CH
CNTW = 1344
CDUM = OWN * R
ADUM = OWN
QCAP = 64
NB = 10
BM = N // NB


def _dense_body(x_ref, w_ref, b_ref, o_ref):
    r = pl.program_id(0)
    acc = jnp.dot(x_ref[...], w_ref[0], preferred_element_type=jnp.float32)
    o_ref[...] = acc + jnp.where(r == R, 1.0, 0.0) * b_ref[...]


def _dense_table(x, w5, bias2d):
    return pl.pallas_call(
        _dense_body,
        grid=(R + 1, NB),
        in_specs=[
            pl.BlockSpec((BM, D), lambda r, i: (i, 0)),
            pl.BlockSpec((1, D, D), lambda r, i: (r, 0, 0)),
            pl.BlockSpec((1, D), lambda r, i: (0, 0)),
        ],
        out_specs=pl.BlockSpec((BM, D), lambda r, i: (r * NB + i, 0)),
        out_shape=jax.ShapeDtypeStruct(((R + 1) * N, D), jnp.float32),
    )(x, w5, bias2d)


def _sc_body(table, srcv, dstv, typv, eww, out,
             src_b, dst_b, typ_b, ew_b, cnt_l, rows,
             qg, qd, qw, gidx_b, acc, sem, msem, gsem):
    core = lax.axis_index("c")
    sub = lax.axis_index("s")
    wid = sub * NCORES + core
    own_start = jnp.minimum(wid * OWN, LASTSTART)
    iota16 = lax.iota(jnp.int32, 16)
    zeros16 = jnp.zeros((16,), jnp.float32)

    def _zero(i, _):
        cnt_l[pl.ds(i * 16, 16)] = zeros16
        return 0
    lax.fori_loop(0, CNTW // 16, _zero, 0)
    for q16 in range(QCAP // 16):
        qg[pl.ds(q16 * 16, 16)] = jnp.zeros((16,), jnp.int32)
    pltpu.sync_copy(table.at[pl.ds(R * N + own_start, OWN)],
                    acc.at[pl.ds(0, OWN)])

    def _meta(ch, par, arrays):
        off = ch * CH
        return [pltpu.make_async_copy(hbm.at[pl.ds(off, CH)], buf.at[par],
                                      msem.at[par]) for hbm, buf in arrays]

    ones = jnp.full((16,), 1.0, jnp.float32)
    p1arrs = [(dstv, dst_b), (typv, typ_b)]
    for c in _meta(0, 0, p1arrs):
        c.start()

    def _p1_chunk(ch, _):
        par = ch & 1

        @pl.when(ch + 1 < NCHUNK)
        def _():
            for c in _meta(ch + 1, 1 - par, p1arrs):
                c.start()
        for c in _meta(ch, par, p1arrs):
            c.wait()

        @plsc.parallel_loop(0, GRP2, 1, unroll=4)
        def _grp(g):
            for h in range(2):
                o = g * 32 + h * 16
                d = dst_b[par, pl.ds(o, 16)]
                t = typ_b[par, pl.ds(o, 16)]
                dl = d - own_start
                m = (dl >= 0) & (dl < OWN)
                cidx = jnp.where(m, dl * R + t, CDUM)
                plsc.addupdate_scatter(cnt_l, [cidx], ones, mask=m)
        return 0
    lax.fori_loop(0, NCHUNK, _p1_chunk, 0)

    def _stage_fire(j, base):
        j32 = pl.multiple_of(j * 32, 32)
        for b in range(2):
            gidx_b[pl.ds(j32 + b * 16, 16)] = qg[pl.ds(base + b * 16, 16)]
            qd[pl.ds(QCAP + j32 + b * 16, 16)] = qd[pl.ds(base + b * 16, 16)]
            qw[pl.ds(QCAP + j32 + b * 16, 16)] = qw[pl.ds(base + b * 16, 16)]
        for b in range(2):
            pltpu.make_async_copy(
                table.at[gidx_b.at[pl.ds(j32 + b * 16, 16)]],
                rows.at[pl.ds(j32 + b * 16, 16)], gsem.at[j]).start()

    def _wait_acc(j):
        j32 = pl.multiple_of(j * 32, 32)
        for b in range(2):
            pltpu.make_async_copy(
                table.at[gidx_b.at[pl.ds(j32 + b * 16, 16)]],
                rows.at[pl.ds(j32 + b * 16, 16)], gsem.at[j]).wait()
        dv = [qd[pl.ds(QCAP + j32 + b * 16, 16)] for b in range(2)]
        sv = [qw[pl.ds(QCAP + j32 + b * 16, 16)] for b in range(2)]

        @plsc.parallel_loop(0, 32, 1, unroll=4)
        def _edges(e):
            lane = iota16 == (e & 15)
            d16 = jnp.where(e < 16, dv[0], dv[1])
            s16 = jnp.where(e < 16, sv[0], sv[1])
            dl_e = jnp.sum(jnp.where(lane, d16, 0))
            sc_e = jnp.sum(jnp.where(lane, s16, 0.0))
            for j in range(D // 16):
                v = rows[j32 + e, pl.ds(j * 16, 16)]
                plsc.addupdate(acc.at[dl_e, pl.ds(j * 16, 16)], v * sc_e)

    p2arrs = [(srcv, src_b), (dstv, dst_b), (typv, typ_b), (eww, ew_b)]
    for c in _meta(0, 0, p2arrs):
        c.start()

    def _p2_chunk(ch, carry):
        par = ch & 1

        @pl.when(ch + 1 < NCHUNK)
        def _():
            for c in _meta(ch + 1, 1 - par, p2arrs):
                c.start()
        for c in _meta(ch, par, p2arrs):
            c.wait()

        @plsc.parallel_loop(0, GRP2, 1, unroll=4)
        def _pre(g):
            for h in range(2):
                o = g * 32 + h * 16
                s = src_b[par, pl.ds(o, 16)]
                d = dst_b[par, pl.ds(o, 16)]
                t = typ_b[par, pl.ds(o, 16)]
                w = ew_b[par, pl.ds(o, 16)]
                dl = d - own_start
                m = (dl >= 0) & (dl < OWN)
                cidx = jnp.where(m, dl * R + t, CDUM)
                cnt = plsc.load_gather(cnt_l, [cidx])
                src_b[par, pl.ds(o, 16)] = t * N + s
                dst_b[par, pl.ds(o, 16)] = dl
                ew_b[par, pl.ds(o, 16)] = w / jnp.maximum(cnt, 1.0)

        def _grp(g, carry):
            qc, pend, pr = carry
            for h in range(2):
                o = g * 32 + h * 16
                gi = src_b[par, pl.ds(o, 16)]
                dl = dst_b[par, pl.ds(o, 16)]
                sc = ew_b[par, pl.ds(o, 16)]
                m = (dl >= 0) & (dl < OWN)
                plsc.store_compressed(qg.at[pl.ds(qc, 16)], gi, mask=m)
                plsc.store_compressed(qd.at[pl.ds(qc, 16)], dl, mask=m)
                plsc.store_compressed(qw.at[pl.ds(qc, 16)], sc, mask=m)
                qc = qc + plsc.all_reduce_population_count(m)[0]
                fire = qc >= 32

                @pl.when(fire)
                def _():
                    _stage_fire(pr, qc - 32)

                    @pl.when(pend == 1)
                    def _():
                        _wait_acc(1 - pr)
                qc = jnp.where(fire, qc - 32, qc)
                pend = jnp.where(fire, 1, pend)
                pr = jnp.where(fire, 1 - pr, pr)
            return (qc, pend, pr)
        return lax.fori_loop(0, GRP2, _grp, carry)
    qc, pend, pr = lax.fori_loop(0, NCHUNK, _p2_chunk,
                                 (0, 0, 0))

    @pl.when(pend == 1)
    def _():
        _wait_acc(1 - pr)

    livecnt0 = jnp.minimum(qc, 16)
    livecnt1 = jnp.clip(qc - 16, 0, 16)
    qd[pl.ds(0, 16)] = jnp.where(iota16 < livecnt0, qd[pl.ds(0, 16)], ADUM)
    qw[pl.ds(0, 16)] = jnp.where(iota16 < livecnt0, qw[pl.ds(0, 16)], 0.0)
    qd[pl.ds(16, 16)] = jnp.where(iota16 < livecnt1, qd[pl.ds(16, 16)], ADUM)
    qw[pl.ds(16, 16)] = jnp.where(iota16 < livecnt1, qw[pl.ds(16, 16)], 0.0)
    _stage_fire(jnp.int32(0), 0)
    _wait_acc(jnp.int32(0))

    pltpu.sync_copy(acc.at[pl.ds(0, OWN)], out.at[pl.ds(own_start, OWN)])


_sc_kernel = functools.partial(
    pl.kernel,
    out_type=jax.ShapeDtypeStruct((N, D), jnp.float32),
    mesh=plsc.VectorSubcoreMesh(core_axis_name="c", subcore_axis_name="s",
                                num_cores=NCORES, num_subcores=NSUB),
    compiler_params=pltpu.CompilerParams(needs_layout_passes=False),
    scratch_types=[
        pltpu.VMEM((2, CH), jnp.int32),
        pltpu.VMEM((2, CH), jnp.int32),
        pltpu.VMEM((2, CH), jnp.int32),
        pltpu.VMEM((2, CH), jnp.float32),
        pltpu.VMEM((CNTW,), jnp.float32),
        pltpu.VMEM((64, D), jnp.float32),
        pltpu.VMEM((QCAP,), jnp.int32),
        pltpu.VMEM((QCAP + 64,), jnp.int32),
        pltpu.VMEM((QCAP + 64,), jnp.float32),
        pltpu.VMEM((64,), jnp.int32),
        pltpu.VMEM((OWN + 8, D), jnp.float32),
        pltpu.SemaphoreType.DMA,
        pltpu.SemaphoreType.DMA((2,)),
        pltpu.SemaphoreType.DMA((2,)),
    ],
)(_sc_body)


def kernel(x, edge_index, edge_type, edge_weight, weight, root, bias):
    x = x.astype(jnp.float32)
    src = edge_index[0].astype(jnp.int32)
    dst = edge_index[1].astype(jnp.int32)
    typ = edge_type.astype(jnp.int32)
    ew = edge_weight.astype(jnp.float32)
    w5 = jnp.concatenate([weight.astype(jnp.float32),
                          root.astype(jnp.float32)[None]], axis=0)
    table = _dense_table(x, w5, bias.astype(jnp.float32).reshape(1, D))
    return _sc_kernel(table, src, dst, typ, ew)
